# baseline (speedup 1.0000x reference)
import jax
import jax.numpy as jnp
import numpy as np
from jax import lax
from jax.experimental import pallas as pl
from jax.experimental.pallas import tpu as pltpu
from jax.experimental.pallas import tpu_sc as plsc

_N = 64
_NSQ = _N * _N
_POOLING_COUNTS = [15, 8, 8]
_D_TILE = 256
_SC_DCHUNK = 256


def _build_mask() -> np.ndarray:
    mask = np.zeros((_N, _N), dtype=bool)
    d = np.arange(_N)
    mask[d, d] = True
    stride, offset = 1, 0
    for c in _POOLING_COUNTS:
        for _ in range(c):
            offset += stride
            i = np.arange(0, _N - offset, stride)
            j = np.arange(offset, _N, stride)
            mask[i, j] = True
        stride *= 2
    return mask


_MASK_NP = _build_mask()

_MM = np.arange(_N)[:, None]
_QI = (np.arange(_NSQ) // _N)[None, :]
_QJ = (np.arange(_NSQ) % _N)[None, :]
_QMID = (_QI + _QJ) // 2
_EQI = (_MM == _QI).astype(np.float32)
_EQJ = (_MM == _QJ).astype(np.float32)
_EQM = (_MM == _QMID).astype(np.float32)
_OH_L = (0.4 * (_EQI + _EQJ) + 0.2 * _EQM).astype(np.float32)
_OH_J = _EQJ


def _mask_from_iota():
    i_idx = jax.lax.broadcasted_iota(jnp.int32, (_N, _N), 0)
    j_idx = jax.lax.broadcasted_iota(jnp.int32, (_N, _N), 1)
    o = j_idx - i_idx
    g0 = (o >= 0) & (o <= 15)
    g1 = (o >= 17) & (o <= 31) & (o % 2 == 1) & (i_idx % 2 == 0)
    g2 = (o >= 35) & (o <= 63) & (o % 4 == 3) & (i_idx % 4 == 0)
    return (g0 | g1 | g2).astype(jnp.float32)


def _attn_kernel(x_ref, qwt_ref, qb_ref, vwt_ref, vb_ref, f_ref):
    xb = x_ref[0]
    xt = xb.T
    m_k = jnp.dot(xt, vwt_ref[...], preferred_element_type=jnp.float32)
    m_k = m_k + vb_ref[0][None, :]
    m_q = jnp.dot(xt, qwt_ref[...], preferred_element_type=jnp.float32)
    m_q = m_q + qb_ref[0][None, :]
    m2m = jax.lax.dot_general(
        m_k, m_q, (((1,), (1,)), ((), ())),
        preferred_element_type=jnp.float32) * 0.125
    m2m = m2m - jnp.max(m2m, axis=-1, keepdims=True)
    e = jnp.exp(m2m)
    w = e / jnp.sum(e, axis=-1, keepdims=True)
    f_ref[...] = (_mask_from_iota() * (1.0 + w))[None, None]


def _maps_kernel(x_ref, f_ref, ohl_ref, l_ref):
    xb = x_ref[0]
    f = f_ref[0]
    l_ref[0] = jnp.dot(xb, ohl_ref[...],
                       preferred_element_type=jnp.float32) * f


def _sc_worker_slice():
    c = lax.axis_index("c")
    s = lax.axis_index("s")
    wid = s * 2 + c
    return wid // 2, (wid % 2) * _SC_DCHUNK


def _sc_ring(compute, out_hbm, b, d0, ob, sems):
    def _loop(t, carry):
        for buf in range(2):
            d = t * 2 + buf
            @pl.when(d >= 2)
            def _():
                pltpu.make_async_copy(
                    ob.at[buf], out_hbm.at[b, d0 + d - 2], sems[buf]).wait()
            compute(d, buf)
            pltpu.make_async_copy(
                ob.at[buf], out_hbm.at[b, d0 + d], sems[buf]).start()
        return carry

    lax.fori_loop(0, _SC_DCHUNK // 2, _loop, 0)
    for buf in range(2):
        dd = d0 + _SC_DCHUNK - 2 + buf
        pltpu.make_async_copy(ob.at[buf], out_hbm.at[b, dd], sems[buf]).wait()


def _sc_boundary_body(x_hbm, f_hbm, out_hbm, x_v, f_v, ob, sem0, sem1):
    b, d0 = _sc_worker_slice()
    pltpu.sync_copy(x_hbm.at[b, pl.ds(d0, _SC_DCHUNK)], x_v)
    pltpu.sync_copy(f_hbm.at[b, 0], f_v)

    def _pre(t, carry):
        f_v[pl.ds(t * 16, 16)] = f_v[pl.ds(t * 16, 16)] * 0.5
        return carry
    lax.fori_loop(0, _NSQ // 16, _pre, 0)

    def _compute(d_local, buf):
        xj = [x_v[d_local, pl.ds(v * 16, 16)] for v in range(4)]
        for i in range(_N):
            xi = xj[i // 16][i % 16]
            for v in range(4):
                o = i * _N + v * 16
                ob[buf, i, pl.ds(v * 16, 16)] = (xi + xj[v]) * f_v[pl.ds(o, 16)]

    _sc_ring(_compute, out_hbm, b, d0, ob, (sem0, sem1))


def _sc_content_body(x_hbm, f_hbm, out_hbm, x_v, f_v, ob, sem0, sem1):
    b, d0 = _sc_worker_slice()
    pltpu.sync_copy(x_hbm.at[b, pl.ds(d0, _SC_DCHUNK)], x_v)
    pltpu.sync_copy(f_hbm.at[b, 0], f_v)
    base16 = lax.iota(jnp.int32, 16)

    def _compute(d_local, buf):
        xj = [x_v[d_local, pl.ds(v * 16, 16)] for v in range(4)]
        rm = [base16 * 0 - 1e30 for _ in range(4)]
        for i in range(_N - 1, -1, -1):
            xi = xj[i // 16][i % 16]
            vsel = i // 16
            rm[vsel] = jnp.where(base16 >= (i % 16),
                                 jnp.maximum(rm[vsel], xi), rm[vsel])
            for v in range(vsel + 1, 4):
                rm[v] = jnp.maximum(rm[v], xi)
            for v in range(4):
                o = i * _N + v * 16
                ob[buf, i, pl.ds(v * 16, 16)] = rm[v] * f_v[pl.ds(o, 16)]

    _sc_ring(_compute, out_hbm, b, d0, ob, (sem0, sem1))


def kernel(x, c_lin_w, c_lin_b, v_lin_w, v_lin_b):
    bsz, dim, n = x.shape
    odim = v_lin_w.shape[0]
    qwt = c_lin_w[:odim].T
    qb = c_lin_b[:odim].reshape(1, odim)
    vwt = v_lin_w.T
    vb = v_lin_b.reshape(1, odim)

    f = pl.pallas_call(
        _attn_kernel,
        grid=(bsz,),
        in_specs=[
            pl.BlockSpec((1, dim, n), lambda b: (b, 0, 0)),
            pl.BlockSpec((dim, odim), lambda b: (0, 0)),
            pl.BlockSpec((1, odim), lambda b: (0, 0)),
            pl.BlockSpec((dim, odim), lambda b: (0, 0)),
            pl.BlockSpec((1, odim), lambda b: (0, 0)),
        ],
        out_specs=pl.BlockSpec((1, 1, n, n), lambda b: (b, 0, 0, 0)),
        out_shape=jax.ShapeDtypeStruct((bsz, 1, n, n), jnp.float32),
    )(x, qwt, qb, vwt, vb)
    f = f.reshape(bsz, 1, _NSQ)

    _sc_scratch = [
        pltpu.VMEM((_SC_DCHUNK, _N), jnp.float32),
        pltpu.VMEM((_NSQ,), jnp.float32),
        pltpu.VMEM((2, _N, _N), jnp.float32),
        pltpu.SemaphoreType.DMA,
        pltpu.SemaphoreType.DMA,
    ]
    _sc_mesh = plsc.VectorSubcoreMesh(core_axis_name="c",
                                      subcore_axis_name="s")
    _sc_params = pltpu.CompilerParams(use_tc_tiling_on_sc=True)
    b_map = pl.kernel(
        _sc_boundary_body,
        out_type=jax.ShapeDtypeStruct((bsz, dim, n, n), jnp.float32),
        mesh=_sc_mesh, scratch_types=_sc_scratch,
        compiler_params=_sc_params)(x, f)
    c_map = pl.kernel(
        _sc_content_body,
        out_type=jax.ShapeDtypeStruct((bsz, dim, n, n), jnp.float32),
        mesh=_sc_mesh, scratch_types=_sc_scratch,
        compiler_params=_sc_params)(x, f)

    dt = _D_TILE
    oh_l = jnp.asarray(_OH_L)
    l_flat = pl.pallas_call(
        _maps_kernel,
        grid=(bsz, dim // dt),
        in_specs=[
            pl.BlockSpec((1, dt, n), lambda b, d: (b, d, 0)),
            pl.BlockSpec((1, 1, _NSQ), lambda b, d: (b, 0, 0)),
            pl.BlockSpec((n, _NSQ), lambda b, d: (0, 0)),
        ],
        out_specs=pl.BlockSpec((1, dt, _NSQ), lambda b, d: (b, d, 0)),
        out_shape=jax.ShapeDtypeStruct((bsz, dim, _NSQ), jnp.float32),
    )(x, f, oh_l)
    l_map = l_flat.reshape(bsz, dim, n, n)

    mask2d = jnp.broadcast_to(
        jnp.asarray(_MASK_NP)[None, None, :, :], (bsz, 1, n, n))
    return (b_map, l_map, c_map, mask2d)


# speedup vs baseline: 1.1087x; 1.1087x over previous
import jax
import jax.numpy as jnp
import numpy as np
from jax import lax
from jax.experimental import pallas as pl
from jax.experimental.pallas import tpu as pltpu
from jax.experimental.pallas import tpu_sc as plsc

_N = 64
_NSQ = _N * _N
_POOLING_COUNTS = [15, 8, 8]
_D_TILE = 256
_SC_DCHUNK = 256


def _build_mask() -> np.ndarray:
    mask = np.zeros((_N, _N), dtype=bool)
    d = np.arange(_N)
    mask[d, d] = True
    stride, offset = 1, 0
    for c in _POOLING_COUNTS:
        for _ in range(c):
            offset += stride
            i = np.arange(0, _N - offset, stride)
            j = np.arange(offset, _N, stride)
            mask[i, j] = True
        stride *= 2
    return mask


_MASK_NP = _build_mask()

_MM = np.arange(_N)[:, None]
_QI = (np.arange(_NSQ) // _N)[None, :]
_QJ = (np.arange(_NSQ) % _N)[None, :]
_QMID = (_QI + _QJ) // 2
_EQI = (_MM == _QI).astype(np.float32)
_EQJ = (_MM == _QJ).astype(np.float32)
_EQM = (_MM == _QMID).astype(np.float32)
_OH_L = (0.4 * (_EQI + _EQJ) + 0.2 * _EQM).astype(np.float32)
_OH_J = _EQJ


def _mask_from_iota():
    i_idx = jax.lax.broadcasted_iota(jnp.int32, (_N, _N), 0)
    j_idx = jax.lax.broadcasted_iota(jnp.int32, (_N, _N), 1)
    o = j_idx - i_idx
    g0 = (o >= 0) & (o <= 15)
    g1 = (o >= 17) & (o <= 31) & (o % 2 == 1) & (i_idx % 2 == 0)
    g2 = (o >= 35) & (o <= 63) & (o % 4 == 3) & (i_idx % 4 == 0)
    return (g0 | g1 | g2).astype(jnp.float32)


def _attn_kernel(x_ref, qwt_ref, qb_ref, vwt_ref, vb_ref, f_ref):
    xb = x_ref[0]
    xt = xb.T
    m_k = jnp.dot(xt, vwt_ref[...], preferred_element_type=jnp.float32)
    m_k = m_k + vb_ref[0][None, :]
    m_q = jnp.dot(xt, qwt_ref[...], preferred_element_type=jnp.float32)
    m_q = m_q + qb_ref[0][None, :]
    m2m = jax.lax.dot_general(
        m_k, m_q, (((1,), (1,)), ((), ())),
        preferred_element_type=jnp.float32) * 0.125
    m2m = m2m - jnp.max(m2m, axis=-1, keepdims=True)
    e = jnp.exp(m2m)
    w = e / jnp.sum(e, axis=-1, keepdims=True)
    f_ref[...] = (_mask_from_iota() * (1.0 + w))[None, None]


def _maps_kernel(x_ref, f_ref, ohl_ref, l_ref):
    xb = x_ref[0]
    f = f_ref[0]
    val = jnp.dot(xb, ohl_ref[...],
                  preferred_element_type=jnp.float32) * f
    l_ref[0] = val.reshape(xb.shape[0], _N, _N)


def _sc_worker_slice():
    c = lax.axis_index("c")
    s = lax.axis_index("s")
    wid = s * 2 + c
    return wid // 2, (wid % 2) * _SC_DCHUNK


def _sc_ring(compute, out_hbm, b, d0, ob, sems):
    def _loop(t, carry):
        for buf in range(2):
            d = t * 2 + buf
            @pl.when(d >= 2)
            def _():
                pltpu.make_async_copy(
                    ob.at[buf], out_hbm.at[b, d0 + d - 2], sems[buf]).wait()
            compute(d, buf)
            pltpu.make_async_copy(
                ob.at[buf], out_hbm.at[b, d0 + d], sems[buf]).start()
        return carry

    lax.fori_loop(0, _SC_DCHUNK // 2, _loop, 0)
    for buf in range(2):
        dd = d0 + _SC_DCHUNK - 2 + buf
        pltpu.make_async_copy(ob.at[buf], out_hbm.at[b, dd], sems[buf]).wait()


def _sc_boundary_body(x_hbm, f_hbm, out_hbm, x_v, f_v, ob, sem0, sem1):
    b, d0 = _sc_worker_slice()
    pltpu.sync_copy(x_hbm.at[b, pl.ds(d0, _SC_DCHUNK)], x_v)
    pltpu.sync_copy(f_hbm.at[b, 0], f_v)

    def _pre(t, carry):
        f_v[pl.ds(t * 16, 16)] = f_v[pl.ds(t * 16, 16)] * 0.5
        return carry
    lax.fori_loop(0, _NSQ // 16, _pre, 0)

    def _compute(d_local, buf):
        xj = [x_v[d_local, pl.ds(v * 16, 16)] for v in range(4)]
        for i in range(_N):
            xi = xj[i // 16][i % 16]
            for v in range(4):
                o = i * _N + v * 16
                ob[buf, pl.ds(o, 16)] = (xi + xj[v]) * f_v[pl.ds(o, 16)]

    _sc_ring(_compute, out_hbm, b, d0, ob, (sem0, sem1))


def _sc_content_body(x_hbm, f_hbm, out_hbm, x_v, f_v, ob, sem0, sem1):
    b, d0 = _sc_worker_slice()
    pltpu.sync_copy(x_hbm.at[b, pl.ds(d0, _SC_DCHUNK)], x_v)
    pltpu.sync_copy(f_hbm.at[b, 0], f_v)
    base16 = lax.iota(jnp.int32, 16)

    def _compute(d_local, buf):
        xj = [x_v[d_local, pl.ds(v * 16, 16)] for v in range(4)]
        rm = [base16 * 0 - 1e30 for _ in range(4)]
        for i in range(_N - 1, -1, -1):
            xi = xj[i // 16][i % 16]
            vsel = i // 16
            rm[vsel] = jnp.where(base16 >= (i % 16),
                                 jnp.maximum(rm[vsel], xi), rm[vsel])
            for v in range(vsel + 1, 4):
                rm[v] = jnp.maximum(rm[v], xi)
            for v in range(4):
                o = i * _N + v * 16
                ob[buf, pl.ds(o, 16)] = rm[v] * f_v[pl.ds(o, 16)]

    _sc_ring(_compute, out_hbm, b, d0, ob, (sem0, sem1))


def kernel(x, c_lin_w, c_lin_b, v_lin_w, v_lin_b):
    bsz, dim, n = x.shape
    odim = v_lin_w.shape[0]
    qwt = c_lin_w[:odim].T
    qb = c_lin_b[:odim].reshape(1, odim)
    vwt = v_lin_w.T
    vb = v_lin_b.reshape(1, odim)

    f = pl.pallas_call(
        _attn_kernel,
        grid=(bsz,),
        in_specs=[
            pl.BlockSpec((1, dim, n), lambda b: (b, 0, 0)),
            pl.BlockSpec((dim, odim), lambda b: (0, 0)),
            pl.BlockSpec((1, odim), lambda b: (0, 0)),
            pl.BlockSpec((dim, odim), lambda b: (0, 0)),
            pl.BlockSpec((1, odim), lambda b: (0, 0)),
        ],
        out_specs=pl.BlockSpec((1, 1, n, n), lambda b: (b, 0, 0, 0)),
        out_shape=jax.ShapeDtypeStruct((bsz, 1, n, n), jnp.float32),
    )(x, qwt, qb, vwt, vb)
    f = f.reshape(bsz, 1, _NSQ)

    _sc_scratch = [
        pltpu.VMEM((_SC_DCHUNK, _N), jnp.float32),
        pltpu.VMEM((_NSQ,), jnp.float32),
        pltpu.VMEM((2, _NSQ), jnp.float32),
        pltpu.SemaphoreType.DMA,
        pltpu.SemaphoreType.DMA,
    ]
    _sc_mesh = plsc.VectorSubcoreMesh(core_axis_name="c",
                                      subcore_axis_name="s")
    b_flat = pl.kernel(
        _sc_boundary_body,
        out_type=jax.ShapeDtypeStruct((bsz, dim, _NSQ), jnp.float32),
        mesh=_sc_mesh, scratch_types=_sc_scratch)(x, f)
    c_flat = pl.kernel(
        _sc_content_body,
        out_type=jax.ShapeDtypeStruct((bsz, dim, _NSQ), jnp.float32),
        mesh=_sc_mesh, scratch_types=_sc_scratch)(x, f)

    dt = _D_TILE
    oh_l = jnp.asarray(_OH_L)
    l_map = pl.pallas_call(
        _maps_kernel,
        grid=(bsz, dim // dt),
        in_specs=[
            pl.BlockSpec((1, dt, n), lambda b, d: (b, d, 0)),
            pl.BlockSpec((1, 1, _NSQ), lambda b, d: (b, 0, 0)),
            pl.BlockSpec((n, _NSQ), lambda b, d: (0, 0)),
        ],
        out_specs=pl.BlockSpec((1, dt, n, n), lambda b, d: (b, d, 0, 0)),
        out_shape=jax.ShapeDtypeStruct((bsz, dim, n, n), jnp.float32),
    )(x, f, oh_l)
    b_map = b_flat.reshape(bsz, dim, n, n)
    c_map = c_flat.reshape(bsz, dim, n, n)

    mask2d = jnp.broadcast_to(
        jnp.asarray(_MASK_NP)[None, None, :, :], (bsz, 1, n, n))
    return (b_map, l_map, c_map, mask2d)


# speedup vs baseline: 1.3494x; 1.2170x over previous
import jax
import jax.numpy as jnp
import numpy as np
from jax import lax
from jax.experimental import pallas as pl
from jax.experimental.pallas import tpu as pltpu
from jax.experimental.pallas import tpu_sc as plsc

_N = 64
_NSQ = _N * _N
_POOLING_COUNTS = [15, 8, 8]
_D_TILE = 256
_SC_DCHUNK = 256


def _build_mask() -> np.ndarray:
    mask = np.zeros((_N, _N), dtype=bool)
    d = np.arange(_N)
    mask[d, d] = True
    stride, offset = 1, 0
    for c in _POOLING_COUNTS:
        for _ in range(c):
            offset += stride
            i = np.arange(0, _N - offset, stride)
            j = np.arange(offset, _N, stride)
            mask[i, j] = True
        stride *= 2
    return mask


_MASK_NP = _build_mask()

_MM = np.arange(_N)[:, None]
_QI = (np.arange(_NSQ) // _N)[None, :]
_QJ = (np.arange(_NSQ) % _N)[None, :]
_QMID = (_QI + _QJ) // 2
_EQI = (_MM == _QI).astype(np.float32)
_EQJ = (_MM == _QJ).astype(np.float32)
_EQM = (_MM == _QMID).astype(np.float32)
_OH_L = (0.4 * (_EQI + _EQJ) + 0.2 * _EQM).astype(np.float32)
_OH_J = _EQJ


def _mask_from_iota():
    i_idx = jax.lax.broadcasted_iota(jnp.int32, (_N, _N), 0)
    j_idx = jax.lax.broadcasted_iota(jnp.int32, (_N, _N), 1)
    o = j_idx - i_idx
    g0 = (o >= 0) & (o <= 15)
    g1 = (o >= 17) & (o <= 31) & (o % 2 == 1) & (i_idx % 2 == 0)
    g2 = (o >= 35) & (o <= 63) & (o % 4 == 3) & (i_idx % 4 == 0)
    return (g0 | g1 | g2).astype(jnp.float32)


def _attn_kernel(x_ref, qwt_ref, qb_ref, vwt_ref, vb_ref, f_ref):
    xb = x_ref[0]
    xt = xb.T
    m_k = jnp.dot(xt, vwt_ref[...], preferred_element_type=jnp.float32)
    m_k = m_k + vb_ref[0][None, :]
    m_q = jnp.dot(xt, qwt_ref[...], preferred_element_type=jnp.float32)
    m_q = m_q + qb_ref[0][None, :]
    m2m = jax.lax.dot_general(
        m_k, m_q, (((1,), (1,)), ((), ())),
        preferred_element_type=jnp.float32) * 0.125
    m2m = m2m - jnp.max(m2m, axis=-1, keepdims=True)
    e = jnp.exp(m2m)
    w = e / jnp.sum(e, axis=-1, keepdims=True)
    f_ref[...] = (_mask_from_iota() * (1.0 + w))[None, None]


def _maps_kernel(x_ref, f_ref, ohl_ref, l_ref):
    xb = x_ref[0]
    f = f_ref[0]
    l_ref[0] = jnp.dot(xb, ohl_ref[...],
                       preferred_element_type=jnp.float32) * f


def _sc_worker_slice():
    c = lax.axis_index("c")
    s = lax.axis_index("s")
    wid = s * 2 + c
    return wid // 2, (wid % 2) * _SC_DCHUNK


def _sc_ring(compute, out_hbm, b, d0, ob, sems):
    def _loop(t, carry):
        for buf in range(2):
            d = t * 2 + buf
            @pl.when(d >= 2)
            def _():
                pltpu.make_async_copy(
                    ob.at[buf], out_hbm.at[b, d0 + d - 2], sems[buf]).wait()
            compute(d, buf)
            pltpu.make_async_copy(
                ob.at[buf], out_hbm.at[b, d0 + d], sems[buf]).start()
        return carry

    lax.fori_loop(0, _SC_DCHUNK // 2, _loop, 0)
    for buf in range(2):
        dd = d0 + _SC_DCHUNK - 2 + buf
        pltpu.make_async_copy(ob.at[buf], out_hbm.at[b, dd], sems[buf]).wait()


def _sc_boundary_body(x_hbm, f_hbm, out_hbm, x_v, f_v, ob, sem0, sem1):
    b, d0 = _sc_worker_slice()
    pltpu.sync_copy(x_hbm.at[b, pl.ds(d0, _SC_DCHUNK)], x_v)
    pltpu.sync_copy(f_hbm.at[b, 0], f_v)

    def _pre(t, carry):
        f_v[pl.ds(t * 16, 16)] = f_v[pl.ds(t * 16, 16)] * 0.5
        return carry
    lax.fori_loop(0, _NSQ // 16, _pre, 0)

    def _compute(d_local, buf):
        xj = [x_v[d_local, pl.ds(v * 16, 16)] for v in range(4)]
        for i in range(_N):
            xi = xj[i // 16][i % 16]
            for v in range(4):
                o = i * _N + v * 16
                ob[buf, pl.ds(o, 16)] = (xi + xj[v]) * f_v[pl.ds(o, 16)]

    _sc_ring(_compute, out_hbm, b, d0, ob, (sem0, sem1))


def _sc_content_body(x_hbm, f_hbm, out_hbm, x_v, f_v, ob, sem0, sem1):
    b, d0 = _sc_worker_slice()
    pltpu.sync_copy(x_hbm.at[b, pl.ds(d0, _SC_DCHUNK)], x_v)
    pltpu.sync_copy(f_hbm.at[b, 0], f_v)
    base16 = lax.iota(jnp.int32, 16)

    def _compute(d_local, buf):
        xj = [x_v[d_local, pl.ds(v * 16, 16)] for v in range(4)]
        rm = [base16 * 0 - 1e30 for _ in range(4)]
        for i in range(_N - 1, -1, -1):
            xi = xj[i // 16][i % 16]
            vsel = i // 16
            rm[vsel] = jnp.where(base16 >= (i % 16),
                                 jnp.maximum(rm[vsel], xi), rm[vsel])
            for v in range(vsel + 1, 4):
                rm[v] = jnp.maximum(rm[v], xi)
            for v in range(4):
                o = i * _N + v * 16
                ob[buf, pl.ds(o, 16)] = rm[v] * f_v[pl.ds(o, 16)]

    _sc_ring(_compute, out_hbm, b, d0, ob, (sem0, sem1))


def kernel(x, c_lin_w, c_lin_b, v_lin_w, v_lin_b):
    bsz, dim, n = x.shape
    odim = v_lin_w.shape[0]
    qwt = c_lin_w[:odim].T
    qb = c_lin_b[:odim].reshape(1, odim)
    vwt = v_lin_w.T
    vb = v_lin_b.reshape(1, odim)

    f = pl.pallas_call(
        _attn_kernel,
        grid=(bsz,),
        in_specs=[
            pl.BlockSpec((1, dim, n), lambda b: (b, 0, 0)),
            pl.BlockSpec((dim, odim), lambda b: (0, 0)),
            pl.BlockSpec((1, odim), lambda b: (0, 0)),
            pl.BlockSpec((dim, odim), lambda b: (0, 0)),
            pl.BlockSpec((1, odim), lambda b: (0, 0)),
        ],
        out_specs=pl.BlockSpec((1, 1, n, n), lambda b: (b, 0, 0, 0)),
        out_shape=jax.ShapeDtypeStruct((bsz, 1, n, n), jnp.float32),
    )(x, qwt, qb, vwt, vb)
    f = f.reshape(bsz, 1, _NSQ)

    _sc_scratch = [
        pltpu.VMEM((_SC_DCHUNK, _N), jnp.float32),
        pltpu.VMEM((_NSQ,), jnp.float32),
        pltpu.VMEM((2, _NSQ), jnp.float32),
        pltpu.SemaphoreType.DMA,
        pltpu.SemaphoreType.DMA,
    ]
    _sc_mesh = plsc.VectorSubcoreMesh(core_axis_name="c",
                                      subcore_axis_name="s")
    b_flat = pl.kernel(
        _sc_boundary_body,
        out_type=jax.ShapeDtypeStruct((bsz, dim, _NSQ), jnp.float32),
        mesh=_sc_mesh, scratch_types=_sc_scratch)(x, f)
    c_flat = pl.kernel(
        _sc_content_body,
        out_type=jax.ShapeDtypeStruct((bsz, dim, _NSQ), jnp.float32),
        mesh=_sc_mesh, scratch_types=_sc_scratch)(x, f)

    dt = _D_TILE
    oh_l = jnp.asarray(_OH_L)
    l_flat = pl.pallas_call(
        _maps_kernel,
        grid=(bsz, dim // dt),
        in_specs=[
            pl.BlockSpec((1, dt, n), lambda b, d: (b, d, 0)),
            pl.BlockSpec((1, 1, _NSQ), lambda b, d: (b, 0, 0)),
            pl.BlockSpec((n, _NSQ), lambda b, d: (0, 0)),
        ],
        out_specs=pl.BlockSpec((1, dt, _NSQ), lambda b, d: (b, d, 0)),
        out_shape=jax.ShapeDtypeStruct((bsz, dim, _NSQ), jnp.float32),
    )(x, f, oh_l)
    b_map = b_flat.reshape(bsz, dim, n, n)
    l_map = l_flat.reshape(bsz, dim, n, n)
    c_map = c_flat.reshape(bsz, dim, n, n)

    mask2d = jnp.broadcast_to(
        jnp.asarray(_MASK_NP)[None, None, :, :], (bsz, 1, n, n))
    return (b_map, l_map, c_map, mask2d)
